# baseline (speedup 1.0000x reference)
import functools

import jax
import jax.numpy as jnp
from jax import lax
from jax.experimental import pallas as pl
from jax.experimental.pallas import tpu as pltpu
from jax.experimental.pallas import tpu_sc as plsc

VOCAB = 49408
D = 512
N_CLS = 1000
CTX_LEN = 77
M = 16
HDR = M + 1
TOKP = 80
L = 16


NB = 7
HB = 2
HR = 8


def _body(tjidx_hbm, tokt_hbm, table_hbm, ctx_hbm,
          emb_hbm, eof_hbm,
          bufs, hbufs, tjidx_v, tok_v, eof_v, gsems, wsems, hgsems, hwsems,
          *, nc, cpw):
    wid = lax.axis_index("s") * nc + lax.axis_index("c")
    base = jnp.minimum(wid * cpw, N_CLS - cpw)

    pltpu.sync_copy(tjidx_hbm.at[pl.ds(wid * (CTX_LEN * cpw), CTX_LEN * cpw)],
                    tjidx_v)
    pltpu.sync_copy(tokt_hbm.at[pl.ds(wid * (TOKP * cpw), TOKP * cpw)], tok_v)

    def src(j):
        return ctx_hbm if 1 <= j < HDR else table_hbm

    def start_gather(j):
        b = j % NB
        pltpu.async_copy(src(j).at[tjidx_v.at[pl.ds(j * cpw, cpw)]],
                         bufs[b], gsems[b])

    def wait_gather(j):
        b = j % NB
        pltpu.make_async_copy(src(j).at[tjidx_v.at[pl.ds(j * cpw, cpw)]],
                              bufs[b], gsems[b]).wait()

    def start_write(j):
        b = j % NB
        pltpu.async_copy(bufs[b], emb_hbm.at[j, pl.ds(base, cpw)], wsems[b])

    def wait_write(j):
        b = j % NB
        pltpu.make_async_copy(bufs[b], emb_hbm.at[j, pl.ds(base, cpw)],
                              wsems[b]).wait()

    def h_gather_parts(c):
        b = c % HB
        return (src(c).at[tjidx_v.at[pl.ds(c * cpw, HR)]], hbufs[b],
                hgsems[b])

    def h_write_parts(c, k):
        b = c % HB
        return (hbufs[b], emb_hbm.at[c, pl.ds(base + HR * k, HR)], hwsems[b])

    for c in range(HDR):
        if c >= HB:
            for k in range(cpw // HR):
                pltpu.make_async_copy(*h_write_parts(c - HB, k)).wait()
        pltpu.async_copy(*h_gather_parts(c))
        pltpu.make_async_copy(*h_gather_parts(c)).wait()
        for k in range(cpw // HR):
            pltpu.async_copy(*h_write_parts(c, k))

    for j in range(HDR, HDR + NB):
        start_gather(j)

    for c in range(HDR - HB, HDR):
        for k in range(cpw // HR):
            pltpu.make_async_copy(*h_write_parts(c, k)).wait()

    for g in range(cpw // L):
        def eof_body(j, mb, g=g):
            m, best = mb
            v = tok_v[pl.ds(j * cpw + g * L, L)]
            gt = v > m
            best = jnp.where(gt, jnp.full((L,), j, jnp.int32), best)
            m = jnp.maximum(m, v)
            return m, best

        m0 = jnp.full((L,), -1, jnp.int32)
        b0 = jnp.zeros((L,), jnp.int32)
        _, best = lax.fori_loop(0, CTX_LEN, eof_body, (m0, b0))
        eof_v[pl.ds(g * L, L)] = best

    pltpu.sync_copy(eof_v, eof_hbm.at[pl.ds(base, cpw)])

    for j in range(HDR, CTX_LEN):
        if j > HDR and j - 1 + NB < CTX_LEN:
            wait_write(j - 1)
            start_gather(j - 1 + NB)
        wait_gather(j)
        start_write(j)
    for j in range(CTX_LEN - NB, CTX_LEN):
        wait_write(j)


def kernel(tokenized_text_prototypes, token_embedding_table, ctx_vectors):
    tokens = tokenized_text_prototypes.astype(jnp.int32)

    info = plsc.get_sparse_core_info()
    nc, ns = info.num_cores, info.num_subcores
    nw = nc * ns
    cpw = ((N_CLS + nw - 1) // nw + 7) // 8 * 8

    base = jnp.minimum(jnp.arange(nw) * cpw, N_CLS - cpw)
    cls = base[:, None] + jnp.arange(cpw)[None, :]

    blk = tokens[cls]
    pref = jnp.broadcast_to(tokens[0, 0], (nw, 1, cpw))
    ctxi = jnp.broadcast_to(jnp.arange(M, dtype=jnp.int32)[None, :, None],
                            (nw, M, cpw))
    sufi = blk[:, :, HDR:].transpose(0, 2, 1)
    tjidx = jnp.concatenate([pref, ctxi, sufi], axis=1)

    tokt = jnp.pad(blk, ((0, 0), (0, 0), (0, TOKP - CTX_LEN)),
                   constant_values=-1).transpose(0, 2, 1)

    mesh = plsc.VectorSubcoreMesh(core_axis_name="c", subcore_axis_name="s",
                                  num_cores=nc, num_subcores=ns)
    fn = pl.kernel(
        functools.partial(_body, nc=nc, cpw=cpw),
        out_type=(
            jax.ShapeDtypeStruct((CTX_LEN, N_CLS, D), jnp.float32),
            jax.ShapeDtypeStruct((N_CLS,), jnp.int32),
        ),
        mesh=mesh,
        scratch_types=[
            [pltpu.VMEM((cpw, D), jnp.float32) for _ in range(NB)],
            [pltpu.VMEM((HR, D), jnp.float32) for _ in range(HB)],
            pltpu.VMEM((CTX_LEN * cpw,), jnp.int32),
            pltpu.VMEM((TOKP * cpw,), jnp.int32),
            pltpu.VMEM((cpw,), jnp.int32),
            [pltpu.SemaphoreType.DMA for _ in range(NB)],
            [pltpu.SemaphoreType.DMA for _ in range(NB)],
            [pltpu.SemaphoreType.DMA for _ in range(HB)],
            [pltpu.SemaphoreType.DMA for _ in range(HB)],
        ],
        compiler_params=pltpu.CompilerParams(use_tc_tiling_on_sc=True),
    )
    emb77, eofs = fn(tjidx.reshape(-1), tokt.reshape(-1),
                     token_embedding_table, ctx_vectors)
    return jnp.transpose(emb77, (1, 0, 2)), eofs


# speedup vs baseline: 1.0042x; 1.0042x over previous
import functools

import jax
import jax.numpy as jnp
from jax import lax
from jax.experimental import pallas as pl
from jax.experimental.pallas import tpu as pltpu
from jax.experimental.pallas import tpu_sc as plsc

VOCAB = 49408
D = 512
N_CLS = 1000
CTX_LEN = 77
M = 16
HDR = M + 1
TOKP = 80
L = 16


NB = 6
HB = 4
HR = 8


def _body(tjidx_hbm, tokt_hbm, table_hbm, ctx_hbm,
          emb_hbm, eof_hbm,
          bufs, hbufs, tjidx_v, tok_v, eof_v, gsems, wsems, hgsems, hwsems,
          *, nc, cpw):
    wid = lax.axis_index("s") * nc + lax.axis_index("c")
    base = jnp.minimum(wid * cpw, N_CLS - cpw)

    pltpu.sync_copy(tjidx_hbm.at[pl.ds(wid * (CTX_LEN * cpw), CTX_LEN * cpw)],
                    tjidx_v)
    pltpu.sync_copy(tokt_hbm.at[pl.ds(wid * (TOKP * cpw), TOKP * cpw)], tok_v)

    def src(j):
        return ctx_hbm if 1 <= j < HDR else table_hbm

    def start_gather(j):
        b = j % NB
        pltpu.async_copy(src(j).at[tjidx_v.at[pl.ds(j * cpw, cpw)]],
                         bufs[b], gsems[b])

    def wait_gather(j):
        b = j % NB
        pltpu.make_async_copy(src(j).at[tjidx_v.at[pl.ds(j * cpw, cpw)]],
                              bufs[b], gsems[b]).wait()

    def start_write(j):
        b = j % NB
        pltpu.async_copy(bufs[b], emb_hbm.at[j, pl.ds(base, cpw)], wsems[b])

    def wait_write(j):
        b = j % NB
        pltpu.make_async_copy(bufs[b], emb_hbm.at[j, pl.ds(base, cpw)],
                              wsems[b]).wait()

    def h_gather_parts(c):
        b = c % HB
        return (src(c).at[tjidx_v.at[pl.ds(c * cpw, HR)]], hbufs[b],
                hgsems[b])

    def h_write_parts(c, k):
        b = c % HB
        return (hbufs[b], emb_hbm.at[c, pl.ds(base + HR * k, HR)], hwsems[b])

    for c in range(HDR):
        if c >= HB:
            for k in range(cpw // HR):
                pltpu.make_async_copy(*h_write_parts(c - HB, k)).wait()
        pltpu.async_copy(*h_gather_parts(c))
        pltpu.make_async_copy(*h_gather_parts(c)).wait()
        for k in range(cpw // HR):
            pltpu.async_copy(*h_write_parts(c, k))

    for j in range(HDR, HDR + NB):
        start_gather(j)

    for c in range(HDR - HB, HDR):
        for k in range(cpw // HR):
            pltpu.make_async_copy(*h_write_parts(c, k)).wait()

    for g in range(cpw // L):
        def eof_body(j, mb, g=g):
            m, best = mb
            v = tok_v[pl.ds(j * cpw + g * L, L)]
            gt = v > m
            best = jnp.where(gt, jnp.full((L,), j, jnp.int32), best)
            m = jnp.maximum(m, v)
            return m, best

        m0 = jnp.full((L,), -1, jnp.int32)
        b0 = jnp.zeros((L,), jnp.int32)
        _, best = lax.fori_loop(0, CTX_LEN, eof_body, (m0, b0))
        eof_v[pl.ds(g * L, L)] = best

    pltpu.sync_copy(eof_v, eof_hbm.at[pl.ds(base, cpw)])

    for j in range(HDR, CTX_LEN):
        if j > HDR and j - 1 + NB < CTX_LEN:
            wait_write(j - 1)
            start_gather(j - 1 + NB)
        wait_gather(j)
        start_write(j)
    for j in range(CTX_LEN - NB, CTX_LEN):
        wait_write(j)


def kernel(tokenized_text_prototypes, token_embedding_table, ctx_vectors):
    tokens = tokenized_text_prototypes.astype(jnp.int32)

    info = plsc.get_sparse_core_info()
    nc, ns = info.num_cores, info.num_subcores
    nw = nc * ns
    cpw = ((N_CLS + nw - 1) // nw + 7) // 8 * 8

    base = jnp.minimum(jnp.arange(nw) * cpw, N_CLS - cpw)
    cls = base[:, None] + jnp.arange(cpw)[None, :]

    blk = tokens[cls]
    pref = jnp.broadcast_to(tokens[0, 0], (nw, 1, cpw))
    ctxi = jnp.broadcast_to(jnp.arange(M, dtype=jnp.int32)[None, :, None],
                            (nw, M, cpw))
    sufi = blk[:, :, HDR:].transpose(0, 2, 1)
    tjidx = jnp.concatenate([pref, ctxi, sufi], axis=1)

    tokt = jnp.pad(blk, ((0, 0), (0, 0), (0, TOKP - CTX_LEN)),
                   constant_values=-1).transpose(0, 2, 1)

    mesh = plsc.VectorSubcoreMesh(core_axis_name="c", subcore_axis_name="s",
                                  num_cores=nc, num_subcores=ns)
    fn = pl.kernel(
        functools.partial(_body, nc=nc, cpw=cpw),
        out_type=(
            jax.ShapeDtypeStruct((CTX_LEN, N_CLS, D), jnp.float32),
            jax.ShapeDtypeStruct((N_CLS,), jnp.int32),
        ),
        mesh=mesh,
        scratch_types=[
            [pltpu.VMEM((cpw, D), jnp.float32) for _ in range(NB)],
            [pltpu.VMEM((HR, D), jnp.float32) for _ in range(HB)],
            pltpu.VMEM((CTX_LEN * cpw,), jnp.int32),
            pltpu.VMEM((TOKP * cpw,), jnp.int32),
            pltpu.VMEM((cpw,), jnp.int32),
            [pltpu.SemaphoreType.DMA for _ in range(NB)],
            [pltpu.SemaphoreType.DMA for _ in range(NB)],
            [pltpu.SemaphoreType.DMA for _ in range(HB)],
            [pltpu.SemaphoreType.DMA for _ in range(HB)],
        ],
        compiler_params=pltpu.CompilerParams(use_tc_tiling_on_sc=True),
    )
    emb77, eofs = fn(tjidx.reshape(-1), tokt.reshape(-1),
                     token_embedding_table, ctx_vectors)
    return jnp.transpose(emb77, (1, 0, 2)), eofs
